# baseline (speedup 1.0000x reference)
import functools

import jax
import jax.numpy as jnp
from jax import lax
from jax.experimental import pallas as pl


def _gat_kernel(h_ref, adj_ref, W0_ref, W1_ref, W2_ref, W3_ref, Ac_ref, out_ref, *, k_nei, head_dim, pad_k, num_heads):
    hb = h_ref[0]
    n = hb.shape[0]
    adjb = adj_ref[0]

    ii = lax.broadcasted_iota(jnp.int32, (n, n), 0)
    jj = lax.broadcasted_iota(jnp.int32, (n, n), 1)
    iilt = ii < jj
    mm_row = lax.broadcasted_iota(jnp.int32, (n, pad_k), 1).astype(jnp.float32)
    valid = lax.broadcasted_iota(jnp.int32, (1, pad_k), 1) < k_nei

    outs = []
    W_refs = (W0_ref, W1_ref, W2_ref, W3_ref)
    for hd in range(num_heads):
        Wh = jnp.dot(hb, W_refs[hd][...], preferred_element_type=jnp.float32)
        a_src = Ac_ref[hd, 0, :].reshape(head_dim, 1)
        a_dst = Ac_ref[hd, 1, :].reshape(head_dim, 1)
        s_src = jnp.dot(Wh, a_src, preferred_element_type=jnp.float32)
        s_dst = jnp.dot(Wh, a_dst, preferred_element_type=jnp.float32)
        s_row = s_dst.reshape(1, n)

        gt = (s_dst > s_row).astype(jnp.float32)
        eqb = ((s_dst == s_row) & iilt).astype(jnp.float32)
        rank = jnp.sum(gt + eqb, axis=0, keepdims=True)
        mask = rank < jnp.float32(k_nei)

        e = s_src + s_row
        e = jnp.where(e >= 0, e, 0.2 * e)
        e_m = jnp.where(mask, e, jnp.float32(-1e30))
        m = jnp.max(e_m, axis=1, keepdims=True)
        p = jnp.where(mask, jnp.exp(e - m), jnp.float32(0.0))
        denom = jnp.sum(p, axis=1, keepdims=True)
        att = (p / denom) * adjb
        outs.append(jnp.dot(att, Wh, preferred_element_type=jnp.float32))
    out_ref[0] = jnp.concatenate(outs, axis=-1)


def kernel(h, adj, W, a):
    B, N, D = h.shape
    H, _, d = W.shape
    k_nei = int(0.1 * N)
    pad_k = ((k_nei + 63) // 64) * 64
    a2 = a.reshape(H, 2, d)
    body = functools.partial(_gat_kernel, k_nei=k_nei, head_dim=d, pad_k=pad_k,
                             num_heads=H)
    out = pl.pallas_call(
        body,
        grid=(B,),
        in_specs=[
            pl.BlockSpec((1, N, D), lambda b: (b, 0, 0)),
            pl.BlockSpec((1, N, N), lambda b: (b, 0, 0)),
            pl.BlockSpec((D, d), lambda b: (0, 0)),
            pl.BlockSpec((D, d), lambda b: (0, 0)),
            pl.BlockSpec((D, d), lambda b: (0, 0)),
            pl.BlockSpec((D, d), lambda b: (0, 0)),
            pl.BlockSpec((H, 2, d), lambda b: (0, 0, 0)),
        ],
        out_specs=pl.BlockSpec((1, N, H * d), lambda b: (b, 0, 0)),
        out_shape=jax.ShapeDtypeStruct((B, N, H * d), jnp.float32),
    )(h, adj, W[0], W[1], W[2], W[3], a2)
    return out
